# Initial kernel scaffold; baseline (speedup 1.0000x reference)
import jax
import jax.numpy as jnp
from jax.experimental import pallas as pl


def kernel(x, experts, gate_w, gate_b):
    raise NotImplementedError("write your pallas kernel here")


# speedup vs baseline: 1.6312x; 1.6312x over previous
import functools

import jax
import jax.numpy as jnp
from jax import lax
from jax.experimental import pallas as pl
from jax.experimental.pallas import tpu as pltpu
from jax.experimental.pallas import tpu_sc as plsc

N, C, H, W, E, D = 8, 192, 56, 56, 16, 64
HW = H * W
K = 4
ROWS = N * E * HW
NW = 32
LOC_PER_W = (N * HW) // NW
CHUNK = 112
NCH = LOC_PER_W // CHUNK
RCH = HW // CHUNK


def _gate_kernel(x_ref, gw_ref, gb_ref, idx_ref, wb_ref):
    n = pl.program_id(0)
    xb = x_ref[0]
    logits = jnp.dot(gw_ref[...], xb, preferred_element_type=jnp.float32)
    logits = logits + gb_ref[...]
    m = jnp.max(logits, axis=0, keepdims=True)
    ex = jnp.exp(logits - m)
    rw = ex / jnp.sum(ex, axis=0, keepdims=True)
    erow = lax.broadcasted_iota(jnp.int32, (E, HW), 0)
    hw = lax.broadcasted_iota(jnp.int32, (1, HW), 1)
    rem = rw
    vals = []
    for j in range(K):
        mj = jnp.max(rem, axis=0, keepdims=True)
        amj = jnp.min(jnp.where(rem >= mj, erow, E), axis=0, keepdims=True)
        vals.append(mj)
        idx_ref[0, pl.ds(j, 1), :] = n * (E * HW) + amj * HW + hw
        rem = jnp.where(erow == amj, -jnp.inf, rem)
    vk = jnp.concatenate(vals, axis=0)
    sel = (lax.broadcasted_iota(jnp.int32, (K, D), 1) // 16
           == lax.broadcasted_iota(jnp.int32, (K, D), 0)).astype(jnp.float32)
    wb = lax.dot_general(vk, sel, (((0,), (0,)), ((), ())),
                         preferred_element_type=jnp.float32)
    wb_ref[0] = wb


_gate = pl.pallas_call(
    _gate_kernel,
    grid=(N,),
    in_specs=[
        pl.BlockSpec((1, C, HW), lambda n: (n, 0, 0)),
        pl.BlockSpec((E, C), lambda n: (0, 0)),
        pl.BlockSpec((E, 1), lambda n: (0, 0)),
    ],
    out_specs=[
        pl.BlockSpec((1, K, HW), lambda n: (n, 0, 0)),
        pl.BlockSpec((1, HW, D), lambda n: (n, 0, 0)),
    ],
    out_shape=[
        jax.ShapeDtypeStruct((N, K, HW), jnp.int32),
        jax.ShapeDtypeStruct((N, HW, D), jnp.float32),
    ],
)


def _sc_body(ef_hbm, idx_hbm, wb_hbm, out_hbm,
             idx_v, wb_v, rows_v, outb_v, sem0, sem1):
    wid = lax.axis_index("s") * 2 + lax.axis_index("c")
    n = wid // 4
    r0 = (wid % 4) * NCH
    hw0 = (wid % 4) * LOC_PER_W
    pltpu.sync_copy(idx_hbm.at[n], idx_v)
    sems = (sem0, sem1)

    def issue(c):
        b = c % 2
        hs = []
        for j in range(K):
            cp = pltpu.make_async_copy(
                ef_hbm.at[idx_v.at[j, r0 + c]], rows_v.at[b, j], sems[b])
            cp.start()
            hs.append(cp)
        cp = pltpu.make_async_copy(
            wb_hbm.at[n, pl.ds(hw0 + c * CHUNK, CHUNK), :], wb_v.at[b], sems[b])
        cp.start()
        hs.append(cp)
        return hs

    pending = {0: issue(0)}
    for c in range(NCH):
        if c + 1 < NCH:
            pending[c + 1] = issue(c + 1)
        for cp in pending.pop(c):
            cp.wait()
        b = c % 2

        def body(l, carry, b=b):
            ws = [wb_v[b, l, pl.ds(j * 16, 16)] for j in range(K)]
            for d in range(D // 16):
                acc = ws[0] * rows_v[b, 0, l, pl.ds(d * 16, 16)]
                for j in range(1, K):
                    acc = acc + ws[j] * rows_v[b, j, l, pl.ds(d * 16, 16)]
                outb_v[b, l, pl.ds(d * 16, 16)] = acc
            return carry

        lax.fori_loop(0, CHUNK, body, 0)
        base = wid * LOC_PER_W + c * CHUNK
        pltpu.sync_copy(outb_v.at[b], out_hbm.at[pl.ds(base, CHUNK)])


@functools.cache
def _sc_combine():
    return pl.kernel(
        _sc_body,
        mesh=plsc.VectorSubcoreMesh(core_axis_name="c", subcore_axis_name="s"),
        compiler_params=pltpu.CompilerParams(use_tc_tiling_on_sc=False),
        out_type=jax.ShapeDtypeStruct((N * HW, D), jnp.float32),
        scratch_types=[
            pltpu.VMEM((K, RCH, CHUNK), jnp.int32),
            pltpu.VMEM((2, CHUNK, D), jnp.float32),
            pltpu.VMEM((2, K, CHUNK, D), jnp.float32),
            pltpu.VMEM((2, CHUNK, D), jnp.float32),
            pltpu.SemaphoreType.DMA,
            pltpu.SemaphoreType.DMA,
        ],
    )


def kernel(x, experts, gate_w, gate_b):
    x3 = x.reshape(N, C, HW)
    idx, wb = _gate(x3, gate_w, gate_b.reshape(E, 1))
    idx4 = idx.reshape(N, K, RCH, CHUNK)
    ef = experts.reshape(ROWS, D)
    out = _sc_combine()(ef, idx4, wb)
    return out.reshape(N, H, W, D)


# speedup vs baseline: 2.5943x; 1.5904x over previous
import functools

import jax
import jax.numpy as jnp
from jax import lax
from jax.experimental import pallas as pl
from jax.experimental.pallas import tpu as pltpu
from jax.experimental.pallas import tpu_sc as plsc

N, C, H, W, E, D = 8, 192, 56, 56, 16, 64
HW = H * W
K = 4
NS = 4
NT = N - NS
NWK = 32
SLABS = HW // 8
SPW = (NS * SLABS) // NWK
CSL = 2
NCH = SPW // CSL


def _gate_kernel(x_ref, gw_ref, gb_ref, wdt_ref, wdb_ref):
    xb = x_ref[0]
    logits = jnp.dot(gw_ref[...], xb, preferred_element_type=jnp.float32)
    logits = logits + gb_ref[...]
    m = jnp.max(logits, axis=0, keepdims=True)
    ex = jnp.exp(logits - m)
    rw = ex / jnp.sum(ex, axis=0, keepdims=True)
    erow = lax.broadcasted_iota(jnp.int32, (E, HW), 0)
    rem = rw
    for _ in range(K):
        mj = jnp.max(rem, axis=0, keepdims=True)
        amj = jnp.min(jnp.where(rem >= mj, erow, E), axis=0, keepdims=True)
        rem = jnp.where(erow == amj, -jnp.inf, rem)
    rwm = jnp.where(rem == -jnp.inf, rw, 0.0)
    eye = (lax.broadcasted_iota(jnp.int32, (E, E), 0)
           == lax.broadcasted_iota(jnp.int32, (E, E), 1)).astype(jnp.float32)
    wdt_ref[0] = lax.dot_general(rwm, eye, (((0,), (0,)), ((), ())),
                                 preferred_element_type=jnp.float32)
    sel = (lax.broadcasted_iota(jnp.int32, (E, E * 16), 1) // 16
           == lax.broadcasted_iota(jnp.int32, (E, E * 16), 0)).astype(jnp.float32)
    wdb_ref[0] = lax.dot_general(rwm, sel, (((0,), (0,)), ((), ())),
                                 preferred_element_type=jnp.float32)


_gate = pl.pallas_call(
    _gate_kernel,
    grid=(N,),
    in_specs=[
        pl.BlockSpec((1, C, HW), lambda n: (n, 0, 0)),
        pl.BlockSpec((E, C), lambda n: (0, 0)),
        pl.BlockSpec((E, 1), lambda n: (0, 0)),
    ],
    out_specs=[
        pl.BlockSpec((1, HW, E), lambda n: (n, 0, 0)),
        pl.BlockSpec((1, HW, E * 16), lambda n: (n, 0, 0)),
    ],
    out_shape=[
        jax.ShapeDtypeStruct((N, HW, E), jnp.float32),
        jax.ShapeDtypeStruct((N, HW, E * 16), jnp.float32),
    ],
)


def _tc_dense_kernel(ex_ref, wdt_ref, out_ref):
    e = pl.program_id(1)
    wdt = wdt_ref[0]
    lane = lax.broadcasted_iota(jnp.int32, (HW, E), 1)
    wcol = jnp.sum(jnp.where(lane == e, wdt, 0.0), axis=1, keepdims=True)
    contrib = jnp.broadcast_to(wcol, (HW, D)) * ex_ref[0, 0]

    @pl.when(e == 0)
    def _():
        out_ref[0] = contrib

    @pl.when(e != 0)
    def _():
        out_ref[0] = out_ref[0] + contrib


_tc_dense = pl.pallas_call(
    _tc_dense_kernel,
    grid=(NT, E),
    in_specs=[
        pl.BlockSpec((1, 1, HW, D), lambda i, e: (NS + i, e, 0, 0)),
        pl.BlockSpec((1, HW, E), lambda i, e: (NS + i, 0, 0)),
    ],
    out_specs=pl.BlockSpec((1, HW, D), lambda i, e: (i, 0, 0)),
    out_shape=jax.ShapeDtypeStruct((NT, HW, D), jnp.float32),
)


def _sc_body(ex_hbm, wdb_hbm, out_hbm, rows_v, wdb_v, outb_v, sem0, sem1):
    wid = lax.axis_index("s") * 2 + lax.axis_index("c")
    n = wid // 8
    q = wid % 8
    slab0 = q * SPW
    loc0 = slab0 * 8
    sems = (sem0, sem1)
    NFULL = NCH + 1

    def copies(c, b, nsl):
        return (
            pltpu.make_async_copy(
                ex_hbm.at[n, :, pl.ds(slab0 + c * CSL, nsl)],
                rows_v.at[b, :, pl.ds(0, nsl)], sems[b]),
            pltpu.make_async_copy(
                wdb_hbm.at[n, pl.ds(loc0 + c * CSL * 8, nsl * 8), :],
                wdb_v.at[b, pl.ds(0, nsl * 8)], sems[b]),
        )

    def issue(c, b, nsl):
        for cp in copies(c, b, nsl):
            cp.start()

    def consume(c, b, nsl):
        for cp in copies(c, b, nsl):
            cp.wait()

        def body(l, carry):
            sl = l // 8
            wi = l % 8
            for d in range(D // 16):
                acc = jnp.zeros((16,), jnp.float32)
                for e in range(E):
                    ws = wdb_v[b, l, pl.ds(e * 16, 16)]
                    acc = acc + ws * rows_v[b, e, sl, wi, pl.ds(d * 16, 16)]
                outb_v[b, l, pl.ds(d * 16, 16)] = acc
            return carry

        lax.fori_loop(0, nsl * 8, body, 0)
        base = n * HW + loc0 + c * CSL * 8
        pltpu.sync_copy(outb_v.at[b, pl.ds(0, nsl * 8)],
                        out_hbm.at[pl.ds(base, nsl * 8)])

    issue(0, 0, CSL)
    issue(1, 1, CSL)

    def ring(i, carry):
        c0 = 2 * i
        consume(c0, 0, CSL)

        @pl.when(c0 + 2 < NCH)
        def _():
            issue(c0 + 2, 0, CSL)

        consume(c0 + 1, 1, CSL)

        @pl.when(c0 + 3 < NCH)
        def _():
            issue(c0 + 3, 1, CSL)

        return carry

    lax.fori_loop(0, NCH // 2, ring, 0)
    issue(NCH, 0, SPW - CSL * NCH)
    consume(NCH, 0, SPW - CSL * NCH)


@functools.cache
def _sc_dense():
    return pl.kernel(
        _sc_body,
        mesh=plsc.VectorSubcoreMesh(core_axis_name="c", subcore_axis_name="s"),
        out_type=jax.ShapeDtypeStruct((NS * HW, D), jnp.float32),
        scratch_types=[
            pltpu.VMEM((2, E, CSL, 8, D), jnp.float32),
            pltpu.VMEM((2, CSL * 8, E * 16), jnp.float32),
            pltpu.VMEM((2, CSL * 8, D), jnp.float32),
            pltpu.SemaphoreType.DMA,
            pltpu.SemaphoreType.DMA,
        ],
    )


def kernel(x, experts, gate_w, gate_b):
    x3 = x.reshape(N, C, HW)
    wdt, wdb = _gate(x3, gate_w, gate_b.reshape(E, 1))
    ex5 = experts.reshape(N, E, SLABS, 8, D)
    out_sc = _sc_dense()(ex5, wdb)
    ex4 = experts.reshape(N, E, HW, D)
    out_tc = _tc_dense(ex4, wdt)
    out = jnp.concatenate(
        [out_sc.reshape(NS, H, W, D), out_tc.reshape(NT, H, W, D)], axis=0)
    return out
